# baseline (device time: 108289 ns/iter reference)
import jax
import jax.numpy as jnp
from jax import lax
from jax.experimental import pallas as pl
from jax.experimental.pallas import tpu as pltpu

N_DEV = 8
B, Sq, Skv = 2, 256, 256
HQ_PER, Dh = 4, 64
D_MODEL = 512
HEAD_BLK = HQ_PER * Dh
WINDOW = 128
SCALE = 0.125


def kernel(x, Wq, K_ext, V_ext, Wo):
    def body(x_ref, wq_ref, k_ref, v_ref, wo_ref, out_ref,
             comm_ref, send_sems, recv_sems):
        my = lax.axis_index("i")
        left = lax.rem(my + N_DEV - 1, N_DEV)
        right = lax.rem(my + 1, N_DEV)

        barrier_sem = pltpu.get_barrier_semaphore()
        for nbr in (left, right):
            pl.semaphore_signal(
                barrier_sem, inc=1,
                device_id=(nbr,), device_id_type=pl.DeviceIdType.MESH,
            )
        pl.semaphore_wait(barrier_sem, 2)

        col0 = my * HEAD_BLK
        qi = lax.broadcasted_iota(jnp.int32, (Sq, Skv), 0)
        ki = lax.broadcasted_iota(jnp.int32, (Sq, Skv), 1)
        mask = jnp.abs(qi - ki) <= WINDOW

        for b in range(B):
            qb = jnp.dot(x_ref[b], wq_ref[:, pl.ds(col0, HEAD_BLK)],
                         preferred_element_type=jnp.float32)
            ctx_parts = []
            for h in range(HQ_PER):
                qh = qb[:, h * Dh:(h + 1) * Dh]
                kh = k_ref[b, :, h, :]
                vh = v_ref[b, :, h, :]
                s = jnp.dot(qh, kh.T, preferred_element_type=jnp.float32)
                s = jnp.where(mask, s * SCALE, -1e9)
                m = jnp.max(s, axis=-1, keepdims=True)
                w = jnp.exp(s - m)
                w = w / jnp.sum(w, axis=-1, keepdims=True)
                ctx_parts.append(
                    jnp.dot(w, vh, preferred_element_type=jnp.float32))
            ctx = jnp.concatenate(ctx_parts, axis=1)
            part = jnp.dot(ctx, wo_ref[pl.ds(col0, HEAD_BLK), :],
                           preferred_element_type=jnp.float32)
            out_ref[b] = part
            comm_ref[0, b] = part

        for hop in range(N_DEV - 1):
            send_slot = hop % 2
            recv_slot = (hop + 1) % 2
            rdma = pltpu.make_async_remote_copy(
                src_ref=comm_ref.at[send_slot],
                dst_ref=comm_ref.at[recv_slot],
                send_sem=send_sems.at[hop],
                recv_sem=recv_sems.at[hop],
                device_id=(right,),
                device_id_type=pl.DeviceIdType.MESH,
            )
            rdma.start()
            rdma.wait()
            for b in range(B):
                out_ref[b] = out_ref[b] + comm_ref[recv_slot, b]

    out_shape = jax.ShapeDtypeStruct((B, Sq, D_MODEL), jnp.float32)
    return pl.pallas_call(
        body,
        out_shape=out_shape,
        in_specs=[pl.BlockSpec(memory_space=pltpu.VMEM)] * 5,
        out_specs=pl.BlockSpec(memory_space=pltpu.VMEM),
        scratch_shapes=[
            pltpu.VMEM((2, B, Sq, D_MODEL), jnp.float32),
            pltpu.SemaphoreType.DMA((N_DEV - 1,)),
            pltpu.SemaphoreType.DMA((N_DEV - 1,)),
        ],
        compiler_params=pltpu.CompilerParams(collective_id=0),
    )(x, Wq, K_ext, V_ext, Wo)


# device time: 44184 ns/iter; 2.4509x vs baseline; 2.4509x over previous
import jax
import jax.numpy as jnp
from jax import lax
from jax.experimental import pallas as pl
from jax.experimental.pallas import tpu as pltpu

N_DEV = 8
B, Sq, Skv = 2, 256, 256
HQ_PER, Dh = 4, 64
D_MODEL = 512
HEAD_BLK = HQ_PER * Dh
ROWS = B * Sq
WINDOW = 128
SCALE = 0.125


def kernel(x, Wq, K_ext, V_ext, Wo):
    def body(x_ref, wq_ref, k_ref, v_ref, wo_ref, out_ref,
             recv_ref, send_sems, recv_sems):
        p = lax.axis_index("i")
        bit0 = p & 1
        bit1 = (p >> 1) & 1
        bit2 = (p >> 2) & 1
        fb = [(bit0 ^ bit1) == 1, bit1 == 1, bit2 == 1]
        prt = [p ^ 1, p ^ 3, p ^ 4]

        barrier_sem = pltpu.get_barrier_semaphore()
        for nbr in prt:
            pl.semaphore_signal(
                barrier_sem, inc=1,
                device_id=(nbr,), device_id_type=pl.DeviceIdType.MESH,
            )
        pl.semaphore_wait(barrier_sem, 3)

        qi = lax.broadcasted_iota(jnp.int32, (Sq, Skv), 0)
        ki = lax.broadcasted_iota(jnp.int32, (Sq, Skv), 1)
        mask = jnp.abs(qi - ki) <= WINDOW
        col0 = p * HEAD_BLK

        def compute_partial(bt):
            xb = x_ref[pl.ds(bt, 1)][0]
            kb = k_ref[pl.ds(bt, 1)][0]
            vb = v_ref[pl.ds(bt, 1)][0]
            qb = jnp.dot(xb, wq_ref[:, pl.ds(col0, HEAD_BLK)],
                         preferred_element_type=jnp.float32)
            ctx_parts = []
            for h in range(HQ_PER):
                qh = qb[:, h * Dh:(h + 1) * Dh]
                s = jnp.dot(qh, kb[:, h, :].T,
                            preferred_element_type=jnp.float32)
                s = jnp.where(mask, s * SCALE, -1e9)
                m = jnp.max(s, axis=-1, keepdims=True)
                w = jnp.exp(s - m)
                w = w / jnp.sum(w, axis=-1, keepdims=True)
                ctx_parts.append(
                    jnp.dot(w, vb[:, h, :], preferred_element_type=jnp.float32))
            ctx = jnp.concatenate(ctx_parts, axis=1)
            part = jnp.dot(ctx, wo_ref[pl.ds(col0, HEAD_BLK), :],
                           preferred_element_type=jnp.float32)
            out_ref[pl.ds(pl.multiple_of(bt * Sq, Sq), Sq), :] = part

        halves = [ROWS // 2, ROWS // 4, ROWS // 8]
        roffs = [0, ROWS // 2, 3 * ROWS // 4]
        lo = jnp.int32(0)
        for s in range(3):
            half = halves[s]
            send_lo = pl.multiple_of(lo + jnp.where(fb[s], 0, half), 64)
            keep_lo = pl.multiple_of(lo + jnp.where(fb[s], half, 0), 64)
            rdma = pltpu.make_async_remote_copy(
                src_ref=out_ref.at[pl.ds(send_lo, half)],
                dst_ref=recv_ref.at[pl.ds(roffs[s], half)],
                send_sem=send_sems.at[s],
                recv_sem=recv_sems.at[s],
                device_id=(prt[s],),
                device_id_type=pl.DeviceIdType.MESH,
            )
            if s == 0:
                b_first = jnp.where(fb[0], 0, 1)
                compute_partial(b_first)
                rdma.start()
                compute_partial(1 - b_first)
            else:
                rdma.start()
            rdma.wait()
            cur = out_ref[pl.ds(keep_lo, half), :]
            rv = recv_ref[roffs[s]:roffs[s] + half, :]
            out_ref[pl.ds(keep_lo, half), :] = cur + rv
            lo = keep_lo

        sz = ROWS // 8
        for j, s in enumerate([2, 1, 0]):
            sem = 3 + j
            partner_lo = pl.multiple_of(jnp.where(fb[s], lo - sz, lo + sz), 64)
            lo = pl.multiple_of(lo, 64)
            send = pltpu.make_async_remote_copy(
                src_ref=out_ref.at[pl.ds(lo, sz)],
                dst_ref=out_ref.at[pl.ds(lo, sz)],
                send_sem=send_sems.at[sem],
                recv_sem=recv_sems.at[sem],
                device_id=(prt[s],),
                device_id_type=pl.DeviceIdType.MESH,
            )
            send.start()
            recv = pltpu.make_async_remote_copy(
                src_ref=out_ref.at[pl.ds(partner_lo, sz)],
                dst_ref=out_ref.at[pl.ds(partner_lo, sz)],
                send_sem=send_sems.at[sem],
                recv_sem=recv_sems.at[sem],
                device_id=(prt[s],),
                device_id_type=pl.DeviceIdType.MESH,
            )
            recv.wait_recv()
            send.wait_send()
            lo = jnp.minimum(lo, partner_lo)
            sz *= 2

    out_shape = jax.ShapeDtypeStruct((ROWS, D_MODEL), jnp.float32)
    res = pl.pallas_call(
        body,
        out_shape=out_shape,
        in_specs=[pl.BlockSpec(memory_space=pltpu.VMEM)] * 5,
        out_specs=pl.BlockSpec(memory_space=pltpu.VMEM),
        scratch_shapes=[
            pltpu.VMEM((7 * ROWS // 8, D_MODEL), jnp.float32),
            pltpu.SemaphoreType.DMA((6,)),
            pltpu.SemaphoreType.DMA((6,)),
        ],
        compiler_params=pltpu.CompilerParams(collective_id=0),
    )(x, Wq, K_ext, V_ext, Wo)
    return res.reshape(B, Sq, D_MODEL)


# device time: 12800 ns/iter; 8.4601x vs baseline; 3.4519x over previous
import jax
import jax.numpy as jnp
from jax import lax
from jax.experimental import pallas as pl
from jax.experimental.pallas import tpu as pltpu

N_DEV = 8
B, Sq, Skv = 2, 256, 256
HQ_PER, Dh = 4, 64
D_MODEL = 512
HEAD_BLK = HQ_PER * Dh
ROWS = B * Sq
HALF_C = D_MODEL // 2
WINDOW = 128
SCALE = 0.125

_HALVES = [ROWS // 2, ROWS // 4, ROWS // 8]
_ROFFS = [0, ROWS // 2, 3 * ROWS // 4]


def kernel(x, Wq, K_ext, V_ext, Wo):
    def body(x_ref, wq_ref, k_ref, v_ref, wo_ref, out_ref,
             recv_ref, send_sems, recv_sems):
        p = lax.axis_index("i")
        bit0 = p & 1
        bit1 = (p >> 1) & 1
        bit2 = (p >> 2) & 1

        barrier_sem = pltpu.get_barrier_semaphore()
        for m in (1, 3, 4):
            pl.semaphore_signal(
                barrier_sem, inc=1,
                device_id=(p ^ m,), device_id_type=pl.DeviceIdType.MESH,
            )
        pl.semaphore_wait(barrier_sem, 3)

        qi = lax.broadcasted_iota(jnp.int32, (Sq, Skv), 0)
        ki = lax.broadcasted_iota(jnp.int32, (Sq, Skv), 1)
        mask = jnp.abs(qi - ki) <= WINDOW
        col0 = p * HEAD_BLK

        def compute_partial(bt):
            xb = x_ref[pl.ds(bt, 1)][0]
            kb = k_ref[pl.ds(bt, 1)][0]
            vb = v_ref[pl.ds(bt, 1)][0]
            qb = jnp.dot(xb, wq_ref[:, pl.ds(col0, HEAD_BLK)],
                         preferred_element_type=jnp.float32)
            ctx_parts = []
            for h in range(HQ_PER):
                qh = qb[:, h * Dh:(h + 1) * Dh]
                s = jnp.dot(qh, kb[:, h, :].T,
                            preferred_element_type=jnp.float32)
                s = jnp.where(mask, s * SCALE, -1e9)
                m = jnp.max(s, axis=-1, keepdims=True)
                w = jnp.exp(s - m)
                w = w / jnp.sum(w, axis=-1, keepdims=True)
                ctx_parts.append(
                    jnp.dot(w, vb[:, h, :], preferred_element_type=jnp.float32))
            ctx = jnp.concatenate(ctx_parts, axis=1)
            part = jnp.dot(ctx, wo_ref[pl.ds(col0, HEAD_BLK), :],
                           preferred_element_type=jnp.float32)
            out_ref[pl.ds(pl.multiple_of(bt * Sq, Sq), Sq), :] = part

        bfA = {"c0": 0, "masks": [1, 3, 4], "sem0": 0,
               "f": [(bit0 ^ bit1) == 1, bit1 == 1, bit2 == 1],
               "lo": jnp.int32(0)}
        bfB = {"c0": HALF_C, "masks": [4, 1, 3], "sem0": 6,
               "f": [bit2 == 1, (bit0 ^ bit1) == 1, bit1 == 1],
               "lo": jnp.int32(0)}

        def rs_start(bf, s):
            half = _HALVES[s]
            send_lo = pl.multiple_of(
                bf["lo"] + jnp.where(bf["f"][s], 0, half), 64)
            bf["keep_lo"] = pl.multiple_of(
                bf["lo"] + jnp.where(bf["f"][s], half, 0), 64)
            d = pltpu.make_async_remote_copy(
                src_ref=out_ref.at[pl.ds(send_lo, half),
                                   pl.ds(bf["c0"], HALF_C)],
                dst_ref=recv_ref.at[pl.ds(_ROFFS[s], half),
                                    pl.ds(bf["c0"], HALF_C)],
                send_sem=send_sems.at[bf["sem0"] + s],
                recv_sem=recv_sems.at[bf["sem0"] + s],
                device_id=(p ^ bf["masks"][s],),
                device_id_type=pl.DeviceIdType.MESH,
            )
            d.start()
            bf["pend"] = d

        def rs_finish(bf, s):
            half = _HALVES[s]
            bf["pend"].wait()
            kl = bf["keep_lo"]
            cur = out_ref[pl.ds(kl, half), pl.ds(bf["c0"], HALF_C)]
            rv = recv_ref[_ROFFS[s]:_ROFFS[s] + half,
                          bf["c0"]:bf["c0"] + HALF_C]
            out_ref[pl.ds(kl, half), pl.ds(bf["c0"], HALF_C)] = cur + rv
            bf["lo"] = kl

        def ag_start(bf, s):
            sz = _HALVES[2 - s]
            mi = 2 - s
            sem = bf["sem0"] + 3 + s
            lo = pl.multiple_of(bf["lo"], 64)
            partner_lo = pl.multiple_of(
                jnp.where(bf["f"][mi], lo - sz, lo + sz), 64)
            send = pltpu.make_async_remote_copy(
                src_ref=out_ref.at[pl.ds(lo, sz), pl.ds(bf["c0"], HALF_C)],
                dst_ref=out_ref.at[pl.ds(lo, sz), pl.ds(bf["c0"], HALF_C)],
                send_sem=send_sems.at[sem],
                recv_sem=recv_sems.at[sem],
                device_id=(p ^ bf["masks"][mi],),
                device_id_type=pl.DeviceIdType.MESH,
            )
            send.start()
            recv = pltpu.make_async_remote_copy(
                src_ref=out_ref.at[pl.ds(partner_lo, sz),
                                   pl.ds(bf["c0"], HALF_C)],
                dst_ref=out_ref.at[pl.ds(partner_lo, sz),
                                   pl.ds(bf["c0"], HALF_C)],
                send_sem=send_sems.at[sem],
                recv_sem=recv_sems.at[sem],
                device_id=(p ^ bf["masks"][mi],),
                device_id_type=pl.DeviceIdType.MESH,
            )
            bf["pend"] = (send, recv)
            bf["lo"] = jnp.minimum(lo, partner_lo)

        def ag_finish(bf):
            send, recv = bf["pend"]
            recv.wait_recv()
            send.wait_send()

        b_first = jnp.where(bfA["f"][0], 0, 1)
        compute_partial(b_first)
        rs_start(bfA, 0)
        compute_partial(1 - b_first)
        rs_start(bfB, 0)
        rs_finish(bfA, 0)
        rs_start(bfA, 1)
        rs_finish(bfB, 0)
        rs_start(bfB, 1)
        rs_finish(bfA, 1)
        rs_start(bfA, 2)
        rs_finish(bfB, 1)
        rs_start(bfB, 2)
        rs_finish(bfA, 2)
        ag_start(bfA, 0)
        rs_finish(bfB, 2)
        ag_start(bfB, 0)
        ag_finish(bfA)
        ag_start(bfA, 1)
        ag_finish(bfB)
        ag_start(bfB, 1)
        ag_finish(bfA)
        ag_start(bfA, 2)
        ag_finish(bfB)
        ag_start(bfB, 2)
        ag_finish(bfA)
        ag_finish(bfB)

    out_shape = jax.ShapeDtypeStruct((ROWS, D_MODEL), jnp.float32)
    res = pl.pallas_call(
        body,
        out_shape=out_shape,
        in_specs=[pl.BlockSpec(memory_space=pltpu.VMEM)] * 5,
        out_specs=pl.BlockSpec(memory_space=pltpu.VMEM),
        scratch_shapes=[
            pltpu.VMEM((7 * ROWS // 8, D_MODEL), jnp.float32),
            pltpu.SemaphoreType.DMA((12,)),
            pltpu.SemaphoreType.DMA((12,)),
        ],
        compiler_params=pltpu.CompilerParams(collective_id=0),
    )(x, Wq, K_ext, V_ext, Wo)
    return res.reshape(B, Sq, D_MODEL)
